# Initial kernel scaffold; baseline (speedup 1.0000x reference)
import jax
import jax.numpy as jnp
from jax.experimental import pallas as pl


def kernel(x, edge_index, W, hy_bias, att, conv_bias):
    raise NotImplementedError("write your pallas kernel here")


# speedup vs baseline: 39.3398x; 39.3398x over previous
import dataclasses
import functools

import jax
import jax.numpy as jnp
from jax import lax
from jax.experimental import pallas as pl
from jax.experimental.pallas import tpu as pltpu
from jax.experimental.pallas import tpu_sc as plsc

MIN_NORM = 1e-15
MAXNORM = 1.0 - 4e-3
HEADS = 4
OUT_PC = 32
D = HEADS * OUT_PC
GC = 144
NC, NS = 2, 16
NW = NC * NS
EB = 80
ZROWS = 16

_HI = jax.lax.Precision.HIGHEST


def _rsum(v):
    return jnp.sum(v * v, axis=-1, keepdims=True)


def _norm(v):
    return jnp.maximum(jnp.sqrt(_rsum(v)), MIN_NORM)


def _artanh(z):
    zc = jnp.clip(z, -1.0 + 1e-7, 1.0 - 1e-7)
    return 0.5 * jnp.log((1.0 + zc) / (1.0 - zc))


def _proj(v):
    n = _norm(v)
    return jnp.where(n > MAXNORM, v / n * MAXNORM, v)


def _expmap0(u):
    n = _norm(u)
    return jnp.tanh(n) * u / n


def _prep_body(x_ref, wt_ref, hyb_ref, a_ref, g_ref, s_ref):
    x = x_ref[...]
    xn = _norm(x)
    mx = jnp.dot(x, wt_ref[...], preferred_element_type=jnp.float32,
                 precision=_HI)
    sq = _rsum(mx)
    mxn = jnp.maximum(jnp.sqrt(sq), MIN_NORM)
    res = jnp.tanh(mxn / xn * _artanh(xn)) * mx / mxn
    res = jnp.where(sq == 0.0, jnp.zeros_like(res), res)
    res = _proj(res)
    hb = _proj(_expmap0(hyb_ref[...]))
    x2 = _rsum(res)
    y2 = _rsum(hb)
    xy = jnp.sum(res * hb, axis=-1, keepdims=True)
    num = (1.0 + 2.0 * xy + y2) * res + (1.0 - x2) * hb
    den = 1.0 + 2.0 * xy + x2 * y2
    h = _proj(num / jnp.maximum(den, MIN_NORM))
    hn = _norm(h)
    logx = _artanh(hn) * h / hn
    s = jnp.dot(logx, a_ref[...], preferred_element_type=jnp.float32,
                precision=_HI)
    s_ref[...] = s
    g_ref[...] = jnp.concatenate(
        [logx, s[:, HEADS:2 * HEADS],
         jnp.zeros((logx.shape[0], GC - D - HEADS), jnp.float32)], axis=1)


def _post_body(acc_ref, g_ref, s_ref, cb_ref, bm_ref, cm_ref, o_ref):
    acc = acc_ref[0] + acc_ref[1]
    num = acc[:, :D]
    den8 = acc[:, D:D + 8]
    s = s_ref[...]
    selfsum = jnp.dot(s, cm_ref[...], preferred_element_type=jnp.float32,
                      precision=_HI)
    selfw = jnp.exp(jnp.maximum(selfsum, 0.2 * selfsum))
    den8 = den8 + selfw
    bm = bm_ref[...]
    den = jnp.dot(den8, bm, preferred_element_type=jnp.float32, precision=_HI)
    sw = jnp.dot(selfw, bm, preferred_element_type=jnp.float32, precision=_HI)
    num = num + sw * g_ref[:, :D]
    out = num / (den + 1e-16) + cb_ref[...]
    out = jnp.maximum(out, 0.0)
    o_ref[...] = _proj(_expmap0(out))


def _sc_edge_body(g_hbm, src_hbm, dst_hbm, s_hbm, out_hbm,
                  src_v, dst_v, rows_v, srows_v, w_v, acc_sh, sem,
                  *, n_nodes, n_blocks, epw):
    cid = lax.axis_index("c")
    sid = lax.axis_index("s")
    wid = sid * NC + cid
    rows_main = (n_nodes // (8 * NS)) * 8
    rows_rem = n_nodes - rows_main * NS
    base_r = sid * rows_main

    @pl.loop(0, 16)
    def _zw(i):
        w_v[pl.ds(i * 16, 16)] = jnp.zeros((16,), jnp.float32)

    @pl.loop(0, ZROWS)
    def _zr(r):
        for q in range(GC // 16):
            rows_v[r, pl.ds(q * 16, 16)] = jnp.zeros((16,), jnp.float32)

    @pl.loop(0, rows_main // ZROWS)
    def _zacc(i):
        pltpu.sync_copy(rows_v.at[pl.ds(0, ZROWS)],
                        acc_sh.at[pl.ds(base_r + i * ZROWS, ZROWS)])

    if rows_rem:
        @pl.when(sid == 0)
        def _zrem():
            pltpu.sync_copy(rows_v.at[pl.ds(0, rows_rem)],
                            acc_sh.at[pl.ds(rows_main * NS, rows_rem)])

    plsc.subcore_barrier()

    ebase = wid * epw
    lane = lax.iota(jnp.int32, 16)

    @pl.loop(0, n_blocks)
    def _blk(b):
        base = ebase + b * EB
        pltpu.sync_copy(src_hbm.at[pl.ds(base, EB)], src_v)
        pltpu.sync_copy(dst_hbm.at[pl.ds(base, EB)], dst_v)
        pltpu.async_copy(g_hbm.at[src_v], rows_v, sem).wait()
        pltpu.async_copy(s_hbm.at[dst_v], srows_v, sem).wait()

        for j in range(EB // 16):
            s16 = src_v[pl.ds(j * 16, 16)]
            d16 = dst_v[pl.ds(j * 16, 16)]
            valid = s16 != d16
            erow = lane + j * 16
            for h in range(HEADS):
                hv = jnp.full((16,), h, jnp.int32)
                a = plsc.load_gather(srows_v, [erow, hv])
                bsc = plsc.load_gather(rows_v, [erow, hv + D])
                al = a + bsc
                al = jnp.maximum(al, 0.2 * al)
                w = jnp.where(valid, jnp.exp(al), 0.0)
                w_v[pl.ds(h * 16, 16)] = w

            @pl.loop(0, 16)
            def _scale(e):
                ei = j * 16 + e
                for q in range(D // 16):
                    wq = plsc.load_gather(
                        w_v, [jnp.full((16,), (q // 2) * 16, jnp.int32) + e])
                    sl = (ei, pl.ds(q * 16, 16))
                    rows_v[sl] = rows_v[sl] * wq
                wd = plsc.load_gather(w_v, [lane * 16 + e])
                rows_v[ei, pl.ds(D, 16)] = wd

        pltpu.sync_copy(rows_v, acc_sh.at[dst_v], add=True)

    plsc.subcore_barrier()
    pltpu.sync_copy(acc_sh.at[pl.ds(base_r, rows_main)],
                    out_hbm.at[cid, pl.ds(base_r, rows_main)])
    if rows_rem:
        @pl.when(sid == 0)
        def _drem():
            pltpu.sync_copy(acc_sh.at[pl.ds(rows_main * NS, rows_rem)],
                            out_hbm.at[cid, pl.ds(rows_main * NS, rows_rem)])


def kernel(x, edge_index, W, hy_bias, att, conv_bias):
    n, d_in = x.shape
    e = edge_index.shape[1]
    blk = 1000
    grid = n // blk

    att_i = att[0, :, :OUT_PC].reshape(-1)
    att_j = att[0, :, OUT_PC:].reshape(-1)
    hid = jnp.arange(D) // OUT_PC
    onehot = jax.nn.one_hot(hid, HEADS, dtype=jnp.float32)
    a_mat = jnp.concatenate(
        [onehot * att_i[:, None], onehot * att_j[:, None],
         jnp.zeros((D, 8), jnp.float32)], axis=1)
    bm = jnp.concatenate([onehot.T, jnp.zeros((4, D), jnp.float32)], axis=0)
    eye4 = jnp.eye(HEADS, dtype=jnp.float32)
    cm = jnp.concatenate(
        [jnp.concatenate([eye4, jnp.zeros((4, 4), jnp.float32)], axis=1),
         jnp.concatenate([eye4, jnp.zeros((4, 4), jnp.float32)], axis=1),
         jnp.zeros((8, 8), jnp.float32)], axis=0)

    g, s = pl.pallas_call(
        _prep_body,
        grid=(grid,),
        in_specs=[
            pl.BlockSpec((blk, d_in), lambda i: (i, 0)),
            pl.BlockSpec((d_in, D), lambda i: (0, 0)),
            pl.BlockSpec((1, D), lambda i: (0, 0)),
            pl.BlockSpec((D, 16), lambda i: (0, 0)),
        ],
        out_specs=[
            pl.BlockSpec((blk, GC), lambda i: (i, 0)),
            pl.BlockSpec((blk, 16), lambda i: (i, 0)),
        ],
        out_shape=[
            jax.ShapeDtypeStruct((n, GC), jnp.float32),
            jax.ShapeDtypeStruct((n, 16), jnp.float32),
        ],
    )(x, W.T, hy_bias[None, :], a_mat)

    src = edge_index[0]
    dst = edge_index[1]

    epw = e // NW
    n_blocks = epw // EB
    mesh = plsc.VectorSubcoreMesh(core_axis_name="c", subcore_axis_name="s",
                                  num_cores=NC, num_subcores=NS)
    cp = pltpu.CompilerParams()
    if "needs_layout_passes" in pltpu.CompilerParams.__dataclass_fields__:
        cp = dataclasses.replace(cp, needs_layout_passes=False)
    if "use_tc_tiling_on_sc" in pltpu.CompilerParams.__dataclass_fields__:
        cp = dataclasses.replace(cp, use_tc_tiling_on_sc=False)
    sc_kernel = pl.kernel(
        functools.partial(_sc_edge_body, n_nodes=n, n_blocks=n_blocks,
                          epw=epw),
        out_type=jax.ShapeDtypeStruct((NC, n, GC), jnp.float32),
        mesh=mesh,
        compiler_params=cp,
        scratch_types=[
            pltpu.VMEM((EB,), jnp.int32),
            pltpu.VMEM((EB,), jnp.int32),
            pltpu.VMEM((EB, GC), jnp.float32),
            pltpu.VMEM((EB, 16), jnp.float32),
            pltpu.VMEM((256,), jnp.float32),
            pltpu.VMEM_SHARED((n, GC), jnp.float32),
            pltpu.SemaphoreType.DMA,
        ],
    )
    acc = sc_kernel(g, src, dst, s)

    out = pl.pallas_call(
        _post_body,
        grid=(grid,),
        in_specs=[
            pl.BlockSpec((NC, blk, GC), lambda i: (0, i, 0)),
            pl.BlockSpec((blk, GC), lambda i: (i, 0)),
            pl.BlockSpec((blk, 16), lambda i: (i, 0)),
            pl.BlockSpec((1, D), lambda i: (0, 0)),
            pl.BlockSpec((8, D), lambda i: (0, 0)),
            pl.BlockSpec((16, 8), lambda i: (0, 0)),
        ],
        out_specs=pl.BlockSpec((blk, D), lambda i: (i, 0)),
        out_shape=jax.ShapeDtypeStruct((n, D), jnp.float32),
    )(acc, g, s, conv_bias[None, :], bm, cm)
    return out


# speedup vs baseline: 54.1762x; 1.3771x over previous
import dataclasses
import functools

import jax
import jax.numpy as jnp
from jax import lax
from jax.experimental import pallas as pl
from jax.experimental.pallas import tpu as pltpu
from jax.experimental.pallas import tpu_sc as plsc

MIN_NORM = 1e-15
MAXNORM = 1.0 - 4e-3
HEADS = 4
OUT_PC = 32
D = HEADS * OUT_PC
GC = 144
NC, NS = 2, 16
NW = NC * NS
EB = 80
PB = 25
ZROWS = 16

_HI = jax.lax.Precision.HIGHEST


def _rsum(v):
    return jnp.sum(v * v, axis=-1, keepdims=True)


def _norm(v):
    return jnp.maximum(jnp.sqrt(_rsum(v)), MIN_NORM)


def _artanh(z):
    zc = jnp.clip(z, -1.0 + 1e-7, 1.0 - 1e-7)
    return 0.5 * jnp.log((1.0 + zc) / (1.0 - zc))


def _proj(v):
    n = _norm(v)
    return jnp.where(n > MAXNORM, v / n * MAXNORM, v)


def _expmap0(u):
    n = _norm(u)
    return jnp.tanh(n) * u / n


def _prep_body(x_ref, wt_ref, hyb_ref, a_ref, g_ref, s_ref):
    x = x_ref[...]
    xn = _norm(x)
    mx = jnp.dot(x, wt_ref[...], preferred_element_type=jnp.float32,
                 precision=_HI)
    sq = _rsum(mx)
    mxn = jnp.maximum(jnp.sqrt(sq), MIN_NORM)
    res = jnp.tanh(mxn / xn * _artanh(xn)) * mx / mxn
    res = jnp.where(sq == 0.0, jnp.zeros_like(res), res)
    res = _proj(res)
    hb = _proj(_expmap0(hyb_ref[...]))
    x2 = _rsum(res)
    y2 = _rsum(hb)
    xy = jnp.sum(res * hb, axis=-1, keepdims=True)
    num = (1.0 + 2.0 * xy + y2) * res + (1.0 - x2) * hb
    den = 1.0 + 2.0 * xy + x2 * y2
    h = _proj(num / jnp.maximum(den, MIN_NORM))
    hn = _norm(h)
    logx = _artanh(hn) * h / hn
    s = jnp.dot(logx, a_ref[...], preferred_element_type=jnp.float32,
                precision=_HI)
    s_ref[...] = s
    g_ref[...] = jnp.concatenate(
        [logx, s[:, HEADS:2 * HEADS],
         jnp.zeros((logx.shape[0], GC - D - HEADS), jnp.float32)], axis=1)


def _post_body(acc_ref, g_ref, s_ref, cb_ref, bm_ref, cm_ref, o_ref):
    acc = acc_ref[0] + acc_ref[1]
    num = acc[:, :D]
    den8 = acc[:, D:D + 8]
    s = s_ref[...]
    selfsum = jnp.dot(s, cm_ref[...], preferred_element_type=jnp.float32,
                      precision=_HI)
    selfw = jnp.exp(jnp.maximum(selfsum, 0.2 * selfsum))
    den8 = den8 + selfw
    bm = bm_ref[...]
    den = jnp.dot(den8, bm, preferred_element_type=jnp.float32, precision=_HI)
    sw = jnp.dot(selfw, bm, preferred_element_type=jnp.float32, precision=_HI)
    num = num + sw * g_ref[:, :D]
    out = num / (den + 1e-16) + cb_ref[...]
    out = jnp.maximum(out, 0.0)
    o_ref[...] = _proj(_expmap0(out))


def _sc_edge_body(g_hbm, src2_hbm, dst2_hbm, s_hbm, out_hbm,
                  src_c, dst_c, rows0, srows0, rows1, srows1, w_v, acc_sh,
                  gsem0, gsem1, *, n_nodes, n_blocks):
    cid = lax.axis_index("c")
    sid = lax.axis_index("s")
    wid = sid * NC + cid
    rows_main = (n_nodes // (8 * NS)) * 8
    rows_rem = n_nodes - rows_main * NS
    base_r = sid * rows_main

    @pl.loop(0, 16)
    def _zw(i):
        w_v[pl.ds(i * 16, 16)] = jnp.zeros((16,), jnp.float32)

    @pl.loop(0, ZROWS)
    def _zr(r):
        for q in range(GC // 16):
            rows0[r, pl.ds(q * 16, 16)] = jnp.zeros((16,), jnp.float32)

    @pl.loop(0, rows_main // ZROWS)
    def _zacc(i):
        pltpu.sync_copy(rows0.at[pl.ds(0, ZROWS)],
                        acc_sh.at[pl.ds(base_r + i * ZROWS, ZROWS)])

    if rows_rem:
        @pl.when(sid == 0)
        def _zrem():
            pltpu.sync_copy(rows0.at[pl.ds(0, rows_rem)],
                            acc_sh.at[pl.ds(rows_main * NS, rows_rem)])

    plsc.subcore_barrier()

    lane = lax.iota(jnp.int32, 16)
    lane16 = lane * 16
    hsplat = [jnp.full((16,), h, jnp.int32) for h in range(HEADS)]
    qsplat = [jnp.full((16,), h * 16, jnp.int32) for h in range(HEADS)]

    def compute_and_scatter(i, r_v, s_v):
        for j in range(EB // 16):
            s16 = src_c[i, pl.ds(j * 16, 16)]
            d16 = dst_c[i, pl.ds(j * 16, 16)]
            valid = s16 != d16
            erow = lane + j * 16
            for h in range(HEADS):
                a = plsc.load_gather(s_v, [erow, hsplat[h]])
                bsc = plsc.load_gather(r_v, [erow, hsplat[h] + D])
                al = a + bsc
                al = jnp.maximum(al, 0.2 * al)
                w = jnp.where(valid, jnp.exp(al), 0.0)
                w_v[pl.ds(h * 16, 16)] = w

            @pl.loop(0, 16)
            def _scale(e):
                ei = j * 16 + e
                for q in range(D // 16):
                    wq = plsc.load_gather(w_v, [qsplat[q // 2] + e])
                    sl = (ei, pl.ds(q * 16, 16))
                    r_v[sl] = r_v[sl] * wq
                wd = plsc.load_gather(w_v, [lane16 + e])
                r_v[ei, pl.ds(D, 16)] = wd

        pltpu.sync_copy(r_v, acc_sh.at[dst_c.at[i]], add=True)

    def step(i, r_a, s_a, g_a, r_b, s_b, g_b):
        @pl.when(i < PB - 1)
        def _pf():
            pltpu.async_copy(g_hbm.at[src_c.at[i + 1]], r_b, g_b)
            pltpu.async_copy(s_hbm.at[dst_c.at[i + 1]], s_b, g_b)

        pltpu.make_async_copy(g_hbm.at[src_c.at[i]], r_a, g_a).wait()
        pltpu.make_async_copy(s_hbm.at[dst_c.at[i]], s_a, g_a).wait()
        compute_and_scatter(i, r_a, s_a)

    wblk = wid * n_blocks

    @pl.loop(0, n_blocks // PB)
    def _chunk(c):
        cb = wblk + c * PB
        pltpu.sync_copy(src2_hbm.at[pl.ds(cb, PB)], src_c)
        pltpu.sync_copy(dst2_hbm.at[pl.ds(cb, PB)], dst_c)
        pltpu.async_copy(g_hbm.at[src_c.at[0]], rows0, gsem0)
        pltpu.async_copy(s_hbm.at[dst_c.at[0]], srows0, gsem0)

        @pl.loop(0, PB)
        def _blk(i):
            @pl.when(i % 2 == 0)
            def _even():
                step(i, rows0, srows0, gsem0, rows1, srows1, gsem1)

            @pl.when(i % 2 == 1)
            def _odd():
                step(i, rows1, srows1, gsem1, rows0, srows0, gsem0)

    plsc.subcore_barrier()
    pltpu.sync_copy(acc_sh.at[pl.ds(base_r, rows_main)],
                    out_hbm.at[cid, pl.ds(base_r, rows_main)])
    if rows_rem:
        @pl.when(sid == 0)
        def _drem():
            pltpu.sync_copy(acc_sh.at[pl.ds(rows_main * NS, rows_rem)],
                            out_hbm.at[cid, pl.ds(rows_main * NS, rows_rem)])


def kernel(x, edge_index, W, hy_bias, att, conv_bias):
    n, d_in = x.shape
    e = edge_index.shape[1]
    blk = 1000
    grid = n // blk

    att_i = att[0, :, :OUT_PC].reshape(-1)
    att_j = att[0, :, OUT_PC:].reshape(-1)
    hid = jnp.arange(D) // OUT_PC
    onehot = jax.nn.one_hot(hid, HEADS, dtype=jnp.float32)
    a_mat = jnp.concatenate(
        [onehot * att_i[:, None], onehot * att_j[:, None],
         jnp.zeros((D, 8), jnp.float32)], axis=1)
    bm = jnp.concatenate([onehot.T, jnp.zeros((4, D), jnp.float32)], axis=0)
    eye4 = jnp.eye(HEADS, dtype=jnp.float32)
    cm = jnp.concatenate(
        [jnp.concatenate([eye4, jnp.zeros((4, 4), jnp.float32)], axis=1),
         jnp.concatenate([eye4, jnp.zeros((4, 4), jnp.float32)], axis=1),
         jnp.zeros((8, 8), jnp.float32)], axis=0)

    g, s = pl.pallas_call(
        _prep_body,
        grid=(grid,),
        in_specs=[
            pl.BlockSpec((blk, d_in), lambda i: (i, 0)),
            pl.BlockSpec((d_in, D), lambda i: (0, 0)),
            pl.BlockSpec((1, D), lambda i: (0, 0)),
            pl.BlockSpec((D, 16), lambda i: (0, 0)),
        ],
        out_specs=[
            pl.BlockSpec((blk, GC), lambda i: (i, 0)),
            pl.BlockSpec((blk, 16), lambda i: (i, 0)),
        ],
        out_shape=[
            jax.ShapeDtypeStruct((n, GC), jnp.float32),
            jax.ShapeDtypeStruct((n, 16), jnp.float32),
        ],
    )(x, W.T, hy_bias[None, :], a_mat)

    src = edge_index[0]
    dst = edge_index[1]

    epw = e // NW
    n_blocks = epw // EB
    mesh = plsc.VectorSubcoreMesh(core_axis_name="c", subcore_axis_name="s",
                                  num_cores=NC, num_subcores=NS)
    cp = pltpu.CompilerParams()
    if "needs_layout_passes" in pltpu.CompilerParams.__dataclass_fields__:
        cp = dataclasses.replace(cp, needs_layout_passes=False)
    if "use_tc_tiling_on_sc" in pltpu.CompilerParams.__dataclass_fields__:
        cp = dataclasses.replace(cp, use_tc_tiling_on_sc=False)
    sc_kernel = pl.kernel(
        functools.partial(_sc_edge_body, n_nodes=n, n_blocks=n_blocks),
        out_type=jax.ShapeDtypeStruct((NC, n, GC), jnp.float32),
        mesh=mesh,
        compiler_params=cp,
        scratch_types=[
            pltpu.VMEM((PB, EB), jnp.int32),
            pltpu.VMEM((PB, EB), jnp.int32),
            pltpu.VMEM((EB, GC), jnp.float32),
            pltpu.VMEM((EB, 16), jnp.float32),
            pltpu.VMEM((EB, GC), jnp.float32),
            pltpu.VMEM((EB, 16), jnp.float32),
            pltpu.VMEM((256,), jnp.float32),
            pltpu.VMEM_SHARED((n, GC), jnp.float32),
            pltpu.SemaphoreType.DMA,
            pltpu.SemaphoreType.DMA,
        ],
    )
    acc = sc_kernel(g, src.reshape(e // EB, EB), dst.reshape(e // EB, EB), s)

    out = pl.pallas_call(
        _post_body,
        grid=(grid,),
        in_specs=[
            pl.BlockSpec((NC, blk, GC), lambda i: (0, i, 0)),
            pl.BlockSpec((blk, GC), lambda i: (i, 0)),
            pl.BlockSpec((blk, 16), lambda i: (i, 0)),
            pl.BlockSpec((1, D), lambda i: (0, 0)),
            pl.BlockSpec((8, D), lambda i: (0, 0)),
            pl.BlockSpec((16, 8), lambda i: (0, 0)),
        ],
        out_specs=pl.BlockSpec((blk, D), lambda i: (i, 0)),
        out_shape=jax.ShapeDtypeStruct((n, D), jnp.float32),
    )(acc, g, s, conv_bias[None, :], bm, cm)
    return out


# speedup vs baseline: 104.9241x; 1.9367x over previous
import dataclasses
import functools

import jax
import jax.numpy as jnp
from jax import lax
from jax.experimental import pallas as pl
from jax.experimental.pallas import tpu as pltpu
from jax.experimental.pallas import tpu_sc as plsc

MIN_NORM = 1e-15
MAXNORM = 1.0 - 4e-3
HEADS = 4
OUT_PC = 32
D = HEADS * OUT_PC
GC = 144
NC, NS = 2, 16
NW = NC * NS
EB = 80
PB = 25
ZROWS = 16
WVSZ = (80 // 16) * 256

_HI = jax.lax.Precision.HIGHEST


def _rsum(v):
    return jnp.sum(v * v, axis=-1, keepdims=True)


def _norm(v):
    return jnp.maximum(jnp.sqrt(_rsum(v)), MIN_NORM)


def _artanh(z):
    zc = jnp.clip(z, -1.0 + 1e-7, 1.0 - 1e-7)
    return 0.5 * jnp.log((1.0 + zc) / (1.0 - zc))


def _proj(v):
    n = _norm(v)
    return jnp.where(n > MAXNORM, v / n * MAXNORM, v)


def _expmap0(u):
    n = _norm(u)
    return jnp.tanh(n) * u / n


def _prep_body(x_ref, wt_ref, hyb_ref, a_ref, g_ref, s_ref):
    x = x_ref[...]
    xn = _norm(x)
    mx = jnp.dot(x, wt_ref[...], preferred_element_type=jnp.float32,
                 precision=_HI)
    sq = _rsum(mx)
    mxn = jnp.maximum(jnp.sqrt(sq), MIN_NORM)
    res = jnp.tanh(mxn / xn * _artanh(xn)) * mx / mxn
    res = jnp.where(sq == 0.0, jnp.zeros_like(res), res)
    res = _proj(res)
    hb = _proj(_expmap0(hyb_ref[...]))
    x2 = _rsum(res)
    y2 = _rsum(hb)
    xy = jnp.sum(res * hb, axis=-1, keepdims=True)
    num = (1.0 + 2.0 * xy + y2) * res + (1.0 - x2) * hb
    den = 1.0 + 2.0 * xy + x2 * y2
    h = _proj(num / jnp.maximum(den, MIN_NORM))
    hn = _norm(h)
    logx = _artanh(hn) * h / hn
    s = jnp.dot(logx, a_ref[...], preferred_element_type=jnp.float32,
                precision=_HI)
    s_ref[...] = s
    g_ref[...] = jnp.concatenate(
        [logx, s[:, HEADS:2 * HEADS],
         jnp.zeros((logx.shape[0], GC - D - HEADS), jnp.float32)], axis=1)


def _post_body(acc_ref, g_ref, s_ref, cb_ref, bm_ref, cm_ref, o_ref):
    acc = acc_ref[0] + acc_ref[1]
    num = acc[:, :D]
    den8 = acc[:, D:D + 8]
    s = s_ref[...]
    selfsum = jnp.dot(s, cm_ref[...], preferred_element_type=jnp.float32,
                      precision=_HI)
    selfw = jnp.exp(jnp.maximum(selfsum, 0.2 * selfsum))
    den8 = den8 + selfw
    bm = bm_ref[...]
    den = jnp.dot(den8, bm, preferred_element_type=jnp.float32, precision=_HI)
    sw = jnp.dot(selfw, bm, preferred_element_type=jnp.float32, precision=_HI)
    num = num + sw * g_ref[:, :D]
    out = num / (den + 1e-16) + cb_ref[...]
    out = jnp.maximum(out, 0.0)
    o_ref[...] = _proj(_expmap0(out))


def _sc_edge_body(g_hbm, src2_hbm, dst2_hbm, s_hbm, out_hbm,
                  src_c, dst_c, rows0, srows0, rows1, srows1, w_v, acc_sh,
                  gsem0, gsem1, *, n_nodes, n_blocks):
    cid = lax.axis_index("c")
    sid = lax.axis_index("s")
    wid = sid * NC + cid
    rows_main = (n_nodes // (8 * NS)) * 8
    rows_rem = n_nodes - rows_main * NS
    base_r = sid * rows_main

    @pl.loop(0, WVSZ // 16)
    def _zw(i):
        w_v[pl.ds(i * 16, 16)] = jnp.zeros((16,), jnp.float32)

    @pl.loop(0, ZROWS)
    def _zr(r):
        for q in range(GC // 16):
            rows0[r, pl.ds(q * 16, 16)] = jnp.zeros((16,), jnp.float32)

    @pl.loop(0, rows_main // ZROWS)
    def _zacc(i):
        pltpu.sync_copy(rows0.at[pl.ds(0, ZROWS)],
                        acc_sh.at[pl.ds(base_r + i * ZROWS, ZROWS)])

    if rows_rem:
        @pl.when(sid == 0)
        def _zrem():
            pltpu.sync_copy(rows0.at[pl.ds(0, rows_rem)],
                            acc_sh.at[pl.ds(rows_main * NS, rows_rem)])

    plsc.subcore_barrier()

    lane = lax.iota(jnp.int32, 16)
    lane16 = lane * 16
    hsplat = [jnp.full((16,), h, jnp.int32) for h in range(HEADS)]
    hsplat_d = [jnp.full((16,), h + D, jnp.int32) for h in range(HEADS)]
    qsplat = [jnp.full((16,), h * 16, jnp.int32) for h in range(HEADS)]

    def compute_and_scatter(i, r_v, s_v):
        for j in range(EB // 16):
            s16 = src_c[i, pl.ds(j * 16, 16)]
            d16 = dst_c[i, pl.ds(j * 16, 16)]
            valid = s16 != d16
            erow = lane + j * 16
            for h in range(HEADS):
                a = plsc.load_gather(s_v, [erow, hsplat[h]])
                bsc = plsc.load_gather(r_v, [erow, hsplat_d[h]])
                al = a + bsc
                al = jnp.maximum(al, 0.2 * al)
                w = jnp.where(valid, jnp.exp(al), 0.0)
                w_v[pl.ds(j * 256 + h * 16, 16)] = w

        @plsc.parallel_loop(0, EB, unroll=4)
        def _scale(e):
            base = ((e >> 4) << 8) + (e & 15)
            bs = jnp.full((16,), base, jnp.int32)
            for h in range(HEADS):
                wh = plsc.load_gather(w_v, [bs + qsplat[h]])
                for q2 in range(2):
                    sl = (e, pl.ds(h * 32 + q2 * 16, 16))
                    r_v[sl] = r_v[sl] * wh
            wd = plsc.load_gather(w_v, [bs + lane16])
            r_v[e, pl.ds(D, 16)] = wd

        pltpu.sync_copy(r_v, acc_sh.at[dst_c.at[i]], add=True)

    def step(i, r_a, s_a, g_a, r_b, s_b, g_b):
        @pl.when(i < PB - 1)
        def _pf():
            pltpu.async_copy(g_hbm.at[src_c.at[i + 1]], r_b, g_b)
            pltpu.async_copy(s_hbm.at[dst_c.at[i + 1]], s_b, g_b)

        pltpu.make_async_copy(g_hbm.at[src_c.at[i]], r_a, g_a).wait()
        pltpu.make_async_copy(s_hbm.at[dst_c.at[i]], s_a, g_a).wait()
        compute_and_scatter(i, r_a, s_a)

    wblk = wid * n_blocks

    @pl.loop(0, n_blocks // PB)
    def _chunk(c):
        cb = wblk + c * PB
        pltpu.sync_copy(src2_hbm.at[pl.ds(cb, PB)], src_c)
        pltpu.sync_copy(dst2_hbm.at[pl.ds(cb, PB)], dst_c)
        pltpu.async_copy(g_hbm.at[src_c.at[0]], rows0, gsem0)
        pltpu.async_copy(s_hbm.at[dst_c.at[0]], srows0, gsem0)

        @pl.loop(0, PB)
        def _blk(i):
            @pl.when(i % 2 == 0)
            def _even():
                step(i, rows0, srows0, gsem0, rows1, srows1, gsem1)

            @pl.when(i % 2 == 1)
            def _odd():
                step(i, rows1, srows1, gsem1, rows0, srows0, gsem0)

    plsc.subcore_barrier()
    pltpu.sync_copy(acc_sh.at[pl.ds(base_r, rows_main)],
                    out_hbm.at[cid, pl.ds(base_r, rows_main)])
    if rows_rem:
        @pl.when(sid == 0)
        def _drem():
            pltpu.sync_copy(acc_sh.at[pl.ds(rows_main * NS, rows_rem)],
                            out_hbm.at[cid, pl.ds(rows_main * NS, rows_rem)])


def kernel(x, edge_index, W, hy_bias, att, conv_bias):
    n, d_in = x.shape
    e = edge_index.shape[1]
    blk = 1000
    grid = n // blk

    att_i = att[0, :, :OUT_PC].reshape(-1)
    att_j = att[0, :, OUT_PC:].reshape(-1)
    hid = jnp.arange(D) // OUT_PC
    onehot = jax.nn.one_hot(hid, HEADS, dtype=jnp.float32)
    a_mat = jnp.concatenate(
        [onehot * att_i[:, None], onehot * att_j[:, None],
         jnp.zeros((D, 8), jnp.float32)], axis=1)
    bm = jnp.concatenate([onehot.T, jnp.zeros((4, D), jnp.float32)], axis=0)
    eye4 = jnp.eye(HEADS, dtype=jnp.float32)
    cm = jnp.concatenate(
        [jnp.concatenate([eye4, jnp.zeros((4, 4), jnp.float32)], axis=1),
         jnp.concatenate([eye4, jnp.zeros((4, 4), jnp.float32)], axis=1),
         jnp.zeros((8, 8), jnp.float32)], axis=0)

    g, s = pl.pallas_call(
        _prep_body,
        grid=(grid,),
        in_specs=[
            pl.BlockSpec((blk, d_in), lambda i: (i, 0)),
            pl.BlockSpec((d_in, D), lambda i: (0, 0)),
            pl.BlockSpec((1, D), lambda i: (0, 0)),
            pl.BlockSpec((D, 16), lambda i: (0, 0)),
        ],
        out_specs=[
            pl.BlockSpec((blk, GC), lambda i: (i, 0)),
            pl.BlockSpec((blk, 16), lambda i: (i, 0)),
        ],
        out_shape=[
            jax.ShapeDtypeStruct((n, GC), jnp.float32),
            jax.ShapeDtypeStruct((n, 16), jnp.float32),
        ],
    )(x, W.T, hy_bias[None, :], a_mat)

    src = edge_index[0]
    dst = edge_index[1]

    epw = e // NW
    n_blocks = epw // EB
    mesh = plsc.VectorSubcoreMesh(core_axis_name="c", subcore_axis_name="s",
                                  num_cores=NC, num_subcores=NS)
    cp = pltpu.CompilerParams()
    if "needs_layout_passes" in pltpu.CompilerParams.__dataclass_fields__:
        cp = dataclasses.replace(cp, needs_layout_passes=False)
    if "use_tc_tiling_on_sc" in pltpu.CompilerParams.__dataclass_fields__:
        cp = dataclasses.replace(cp, use_tc_tiling_on_sc=False)
    sc_kernel = pl.kernel(
        functools.partial(_sc_edge_body, n_nodes=n, n_blocks=n_blocks),
        out_type=jax.ShapeDtypeStruct((NC, n, GC), jnp.float32),
        mesh=mesh,
        compiler_params=cp,
        scratch_types=[
            pltpu.VMEM((PB, EB), jnp.int32),
            pltpu.VMEM((PB, EB), jnp.int32),
            pltpu.VMEM((EB, GC), jnp.float32),
            pltpu.VMEM((EB, 16), jnp.float32),
            pltpu.VMEM((EB, GC), jnp.float32),
            pltpu.VMEM((EB, 16), jnp.float32),
            pltpu.VMEM((WVSZ,), jnp.float32),
            pltpu.VMEM_SHARED((n, GC), jnp.float32),
            pltpu.SemaphoreType.DMA,
            pltpu.SemaphoreType.DMA,
        ],
    )
    acc = sc_kernel(g, src.reshape(e // EB, EB), dst.reshape(e // EB, EB), s)

    out = pl.pallas_call(
        _post_body,
        grid=(grid,),
        in_specs=[
            pl.BlockSpec((NC, blk, GC), lambda i: (0, i, 0)),
            pl.BlockSpec((blk, GC), lambda i: (i, 0)),
            pl.BlockSpec((blk, 16), lambda i: (i, 0)),
            pl.BlockSpec((1, D), lambda i: (0, 0)),
            pl.BlockSpec((8, D), lambda i: (0, 0)),
            pl.BlockSpec((16, 8), lambda i: (0, 0)),
        ],
        out_specs=pl.BlockSpec((blk, D), lambda i: (i, 0)),
        out_shape=jax.ShapeDtypeStruct((n, D), jnp.float32),
    )(acc, g, s, conv_bias[None, :], bm, cm)
    return out


# speedup vs baseline: 107.8024x; 1.0274x over previous
import dataclasses
import functools

import jax
import jax.numpy as jnp
from jax import lax
from jax.experimental import pallas as pl
from jax.experimental.pallas import tpu as pltpu
from jax.experimental.pallas import tpu_sc as plsc

MIN_NORM = 1e-15
MAXNORM = 1.0 - 4e-3
HEADS = 4
OUT_PC = 32
D = HEADS * OUT_PC
GC = 144
NC, NS = 2, 16
NW = NC * NS
EB = 80
PB = 25
ZROWS = 16
WVSZ = (80 // 16) * 256

_HI = jax.lax.Precision.HIGHEST


def _rsum(v):
    return jnp.sum(v * v, axis=-1, keepdims=True)


def _norm(v):
    return jnp.maximum(jnp.sqrt(_rsum(v)), MIN_NORM)


def _artanh(z):
    zc = jnp.clip(z, -1.0 + 1e-7, 1.0 - 1e-7)
    return 0.5 * jnp.log((1.0 + zc) / (1.0 - zc))


def _proj(v):
    n = _norm(v)
    return v * jnp.where(n > MAXNORM, MAXNORM / n, 1.0)


def _expmap0(u):
    n = _norm(u)
    return u * (jnp.tanh(n) / n)


def _prep_body(x_ref, wt_ref, hyb_ref, a_ref, g_ref, s_ref):
    x = x_ref[...]
    xn = _norm(x)
    mx = jnp.dot(x, wt_ref[...], preferred_element_type=jnp.float32,
                 precision=_HI)
    sq = _rsum(mx)
    mxn = jnp.maximum(jnp.sqrt(sq), MIN_NORM)
    scale = jnp.tanh(mxn / xn * _artanh(xn)) / mxn
    scale = jnp.where(sq == 0.0, 0.0, scale)
    res = _proj(mx * scale)
    hb = _proj(_expmap0(hyb_ref[...]))
    x2 = _rsum(res)
    y2 = _rsum(hb)
    xy = jnp.sum(res * hb, axis=-1, keepdims=True)
    den_inv = 1.0 / jnp.maximum(1.0 + 2.0 * xy + x2 * y2, MIN_NORM)
    num = ((1.0 + 2.0 * xy + y2) * den_inv) * res + ((1.0 - x2) * den_inv) * hb
    h = _proj(num)
    hn = _norm(h)
    logx = h * (_artanh(hn) / hn)
    s = jnp.dot(logx, a_ref[...], preferred_element_type=jnp.float32,
                precision=_HI)
    s_ref[...] = s
    g_ref[...] = jnp.concatenate(
        [logx, s[:, HEADS:2 * HEADS],
         jnp.zeros((logx.shape[0], GC - D - HEADS), jnp.float32)], axis=1)


def _post_body(acc_ref, g_ref, s_ref, cb_ref, bm_ref, cm_ref, o_ref):
    acc = acc_ref[0] + acc_ref[1]
    num = acc[:, :D]
    den8 = acc[:, D:D + 8]
    s = s_ref[...]
    selfsum = jnp.dot(s, cm_ref[...], preferred_element_type=jnp.float32,
                      precision=_HI)
    selfw = jnp.exp(jnp.maximum(selfsum, 0.2 * selfsum))
    deninv8 = 1.0 / (den8 + selfw + 1e-16)
    bm = bm_ref[...]
    deninv = jnp.dot(deninv8, bm, preferred_element_type=jnp.float32,
                     precision=_HI)
    sw = jnp.dot(selfw, bm, preferred_element_type=jnp.float32, precision=_HI)
    num = num + sw * g_ref[:, :D]
    out = num * deninv + cb_ref[...]
    out = jnp.maximum(out, 0.0)
    o_ref[...] = _proj(_expmap0(out))


def _sc_edge_body(g_hbm, e3_hbm, s_hbm, out_hbm,
                  src_c, dst_c, rows0, srows0, rows1, srows1, w_v, acc_sh,
                  gsem0, gsem1, *, n_nodes, n_blocks):
    cid = lax.axis_index("c")
    sid = lax.axis_index("s")
    wid = sid * NC + cid
    rows_main = (n_nodes // (8 * NS)) * 8
    rows_rem = n_nodes - rows_main * NS
    base_r = sid * rows_main

    @pl.loop(0, WVSZ // 16)
    def _zw(i):
        w_v[pl.ds(i * 16, 16)] = jnp.zeros((16,), jnp.float32)

    @pl.loop(0, ZROWS)
    def _zr(r):
        for q in range(GC // 16):
            rows0[r, pl.ds(q * 16, 16)] = jnp.zeros((16,), jnp.float32)

    @pl.loop(0, rows_main // ZROWS)
    def _zacc(i):
        pltpu.sync_copy(rows0.at[pl.ds(0, ZROWS)],
                        acc_sh.at[pl.ds(base_r + i * ZROWS, ZROWS)])

    if rows_rem:
        @pl.when(sid == 0)
        def _zrem():
            pltpu.sync_copy(rows0.at[pl.ds(0, rows_rem)],
                            acc_sh.at[pl.ds(rows_main * NS, rows_rem)])

    plsc.subcore_barrier()

    lane = lax.iota(jnp.int32, 16)
    lane16 = lane * 16
    hsplat = [jnp.full((16,), h, jnp.int32) for h in range(HEADS)]
    hsplat_d = [jnp.full((16,), h + D, jnp.int32) for h in range(HEADS)]
    qsplat = [jnp.full((16,), h * 16, jnp.int32) for h in range(HEADS)]

    def compute_and_scatter(i, r_v, s_v):
        for j in range(EB // 16):
            s16 = src_c[i, pl.ds(j * 16, 16)]
            d16 = dst_c[i, pl.ds(j * 16, 16)]
            valid = s16 != d16
            erow = lane + j * 16
            for h in range(HEADS):
                a = plsc.load_gather(s_v, [erow, hsplat[h]])
                bsc = plsc.load_gather(r_v, [erow, hsplat_d[h]])
                al = a + bsc
                al = jnp.maximum(al, 0.2 * al)
                w = jnp.where(valid, jnp.exp(al), 0.0)
                w_v[pl.ds(j * 256 + h * 16, 16)] = w

        @plsc.parallel_loop(0, EB, unroll=8)
        def _scale(e):
            base = ((e >> 4) << 8) + (e & 15)
            bs = jnp.full((16,), base, jnp.int32)
            for h in range(HEADS):
                wh = plsc.load_gather(w_v, [bs + qsplat[h]])
                for q2 in range(2):
                    sl = (e, pl.ds(h * 32 + q2 * 16, 16))
                    r_v[sl] = r_v[sl] * wh
            wd = plsc.load_gather(w_v, [bs + lane16])
            r_v[e, pl.ds(D, 16)] = wd

        pltpu.sync_copy(r_v, acc_sh.at[dst_c.at[i]], add=True)

    def step(i, r_a, s_a, g_a, r_b, s_b, g_b):
        @pl.when(i < PB - 1)
        def _pf():
            pltpu.async_copy(g_hbm.at[src_c.at[i + 1]], r_b, g_b)
            pltpu.async_copy(s_hbm.at[dst_c.at[i + 1]], s_b, g_b)

        pltpu.make_async_copy(g_hbm.at[src_c.at[i]], r_a, g_a).wait()
        pltpu.make_async_copy(s_hbm.at[dst_c.at[i]], s_a, g_a).wait()
        compute_and_scatter(i, r_a, s_a)

    wblk = wid * n_blocks

    @pl.loop(0, n_blocks // PB)
    def _chunk(c):
        cb = wblk + c * PB
        pltpu.sync_copy(e3_hbm.at[0, pl.ds(cb, PB)], src_c)
        pltpu.sync_copy(e3_hbm.at[1, pl.ds(cb, PB)], dst_c)
        pltpu.async_copy(g_hbm.at[src_c.at[0]], rows0, gsem0)
        pltpu.async_copy(s_hbm.at[dst_c.at[0]], srows0, gsem0)

        @pl.loop(0, PB)
        def _blk(i):
            @pl.when(i % 2 == 0)
            def _even():
                step(i, rows0, srows0, gsem0, rows1, srows1, gsem1)

            @pl.when(i % 2 == 1)
            def _odd():
                step(i, rows1, srows1, gsem1, rows0, srows0, gsem0)

    plsc.subcore_barrier()
    pltpu.sync_copy(acc_sh.at[pl.ds(base_r, rows_main)],
                    out_hbm.at[cid, pl.ds(base_r, rows_main)])
    if rows_rem:
        @pl.when(sid == 0)
        def _drem():
            pltpu.sync_copy(acc_sh.at[pl.ds(rows_main * NS, rows_rem)],
                            out_hbm.at[cid, pl.ds(rows_main * NS, rows_rem)])


def kernel(x, edge_index, W, hy_bias, att, conv_bias):
    n, d_in = x.shape
    e = edge_index.shape[1]
    blk = 1000
    grid = n // blk

    att_i = att[0, :, :OUT_PC].reshape(-1)
    att_j = att[0, :, OUT_PC:].reshape(-1)
    hid = jnp.arange(D) // OUT_PC
    onehot = jax.nn.one_hot(hid, HEADS, dtype=jnp.float32)
    a_mat = jnp.concatenate(
        [onehot * att_i[:, None], onehot * att_j[:, None],
         jnp.zeros((D, 8), jnp.float32)], axis=1)
    bm = jnp.concatenate([onehot.T, jnp.zeros((4, D), jnp.float32)], axis=0)
    eye4 = jnp.eye(HEADS, dtype=jnp.float32)
    cm = jnp.concatenate(
        [jnp.concatenate([eye4, jnp.zeros((4, 4), jnp.float32)], axis=1),
         jnp.concatenate([eye4, jnp.zeros((4, 4), jnp.float32)], axis=1),
         jnp.zeros((8, 8), jnp.float32)], axis=0)

    g, s = pl.pallas_call(
        _prep_body,
        grid=(grid,),
        in_specs=[
            pl.BlockSpec((blk, d_in), lambda i: (i, 0)),
            pl.BlockSpec((d_in, D), lambda i: (0, 0)),
            pl.BlockSpec((1, D), lambda i: (0, 0)),
            pl.BlockSpec((D, 16), lambda i: (0, 0)),
        ],
        out_specs=[
            pl.BlockSpec((blk, GC), lambda i: (i, 0)),
            pl.BlockSpec((blk, 16), lambda i: (i, 0)),
        ],
        out_shape=[
            jax.ShapeDtypeStruct((n, GC), jnp.float32),
            jax.ShapeDtypeStruct((n, 16), jnp.float32),
        ],
    )(x, W.T, hy_bias[None, :], a_mat)


    epw = e // NW
    n_blocks = epw // EB
    mesh = plsc.VectorSubcoreMesh(core_axis_name="c", subcore_axis_name="s",
                                  num_cores=NC, num_subcores=NS)
    cp = pltpu.CompilerParams()
    if "needs_layout_passes" in pltpu.CompilerParams.__dataclass_fields__:
        cp = dataclasses.replace(cp, needs_layout_passes=False)
    if "use_tc_tiling_on_sc" in pltpu.CompilerParams.__dataclass_fields__:
        cp = dataclasses.replace(cp, use_tc_tiling_on_sc=False)
    sc_kernel = pl.kernel(
        functools.partial(_sc_edge_body, n_nodes=n, n_blocks=n_blocks),
        out_type=jax.ShapeDtypeStruct((NC, n, GC), jnp.float32),
        mesh=mesh,
        compiler_params=cp,
        scratch_types=[
            pltpu.VMEM((PB, EB), jnp.int32),
            pltpu.VMEM((PB, EB), jnp.int32),
            pltpu.VMEM((EB, GC), jnp.float32),
            pltpu.VMEM((EB, 16), jnp.float32),
            pltpu.VMEM((EB, GC), jnp.float32),
            pltpu.VMEM((EB, 16), jnp.float32),
            pltpu.VMEM((WVSZ,), jnp.float32),
            pltpu.VMEM_SHARED((n, GC), jnp.float32),
            pltpu.SemaphoreType.DMA,
            pltpu.SemaphoreType.DMA,
        ],
    )
    acc = sc_kernel(g, edge_index.reshape(2, e // EB, EB), s)

    out = pl.pallas_call(
        _post_body,
        grid=(grid,),
        in_specs=[
            pl.BlockSpec((NC, blk, GC), lambda i: (0, i, 0)),
            pl.BlockSpec((blk, GC), lambda i: (i, 0)),
            pl.BlockSpec((blk, 16), lambda i: (i, 0)),
            pl.BlockSpec((1, D), lambda i: (0, 0)),
            pl.BlockSpec((8, D), lambda i: (0, 0)),
            pl.BlockSpec((16, 8), lambda i: (0, 0)),
        ],
        out_specs=pl.BlockSpec((blk, D), lambda i: (i, 0)),
        out_shape=jax.ShapeDtypeStruct((n, D), jnp.float32),
    )(acc, g, s, conv_bias[None, :], bm, cm)
    return out
